# Initial kernel scaffold; baseline (speedup 1.0000x reference)
import jax
import jax.numpy as jnp
from jax.experimental import pallas as pl


def kernel(descriptors, elems, W, b):
    raise NotImplementedError("write your pallas kernel here")


# speedup vs baseline: 3.0731x; 3.0731x over previous
import functools

import jax
import jax.numpy as jnp
from jax import lax
from jax.experimental import pallas as pl
from jax.experimental.pallas import tpu as pltpu
from jax.experimental.pallas import tpu_sc as plsc

N = 100000
D = 128
N_TYPES = 4
L = 16
BLK = 400
NB = N // BLK
NW = 32
GROUPS = BLK // L


def _body(desc_hbm, elems_hbm, w_hbm, b_hbm, out_hbm,
          wv, bv, desc_v, elems_v, out_v):
    c = lax.axis_index("c")
    s = lax.axis_index("s")
    wid = s * 2 + c
    lo = (wid * NB) // NW
    hi = ((wid + 1) * NB) // NW

    pltpu.sync_copy(w_hbm, wv)
    pltpu.sync_copy(b_hbm, bv)

    lane = lax.broadcasted_iota(jnp.int32, (L,), 0)
    bvec = bv[...]

    def block_body(g, carry):
        base = g * BLK
        pltpu.sync_copy(desc_hbm.at[pl.ds(base * D, BLK * D)], desc_v)
        pltpu.sync_copy(elems_hbm.at[pl.ds(base, BLK)], elems_v)

        def group_body(gi, carry2):
            rowbase = (gi * L + lane) * D
            e16 = elems_v[pl.ds(gi * L, L)]
            ebase = e16 * D
            acc = bvec
            for k in range(D):
                kk = jnp.full((L,), k, jnp.int32)
                dk = plsc.load_gather(desc_v, [rowbase + kk])
                wk = plsc.load_gather(wv, [ebase + kk])
                acc = acc + dk * wk
            out_v[pl.ds(gi * L, L)] = acc
            return carry2

        lax.fori_loop(0, GROUPS, group_body, 0)
        pltpu.sync_copy(out_v, out_hbm.at[pl.ds(base, BLK)])
        return carry

    lax.fori_loop(lo, hi, block_body, 0)


def kernel(descriptors, elems, W, b):
    wf = W.reshape(N_TYPES * D)
    b16 = jnp.broadcast_to(b.astype(jnp.float32), (L,))
    e32 = elems.astype(jnp.int32)
    df = descriptors.reshape(N * D)
    mesh = plsc.VectorSubcoreMesh(core_axis_name="c", subcore_axis_name="s")
    out = pl.kernel(
        _body,
        mesh=mesh,
        out_type=jax.ShapeDtypeStruct((N,), jnp.float32),
        compiler_params=pltpu.CompilerParams(needs_layout_passes=False),
        scratch_types=[
            pltpu.VMEM((N_TYPES * D,), jnp.float32),
            pltpu.VMEM((L,), jnp.float32),
            pltpu.VMEM((BLK * D,), jnp.float32),
            pltpu.VMEM((BLK,), jnp.int32),
            pltpu.VMEM((BLK,), jnp.float32),
        ],
    )(df, e32, wf, b16)
    return out.reshape(N, 1)
